# baseline (speedup 1.0000x reference)
import jax
import jax.numpy as jnp
from jax import lax
from jax.experimental import pallas as pl
from jax.experimental.pallas import tpu as pltpu
from jax.experimental.pallas import tpu_sc as plsc

N_NODE = 10000
N_USER = 6000
D = 128
DP = D // 2
NE = 320000
NM = 2 * NE
NTILE = 16
NC = 2
MPT = NE // NTILE
CK = 80
NSUB = 25
SCM = CK * NSUB
NSUP = MPT // SCM
SPB = 2
NBLK = NSUP // SPB
NB = 3
RU = 4
RUP = 2
CKH = CK // 2
ROWS_PT = N_NODE // NTILE
NOUT = 8192
OPT = NOUT // (NC * NTILE)
OCK = 32
CW_HI = 313
CW_LO = 312
MSK16 = 0xFFFF
MSKHI = -65536

_params = pltpu.CompilerParams(use_tc_tiling_on_sc=False,
                               needs_layout_passes=False)
_mesh = plsc.VectorSubcoreMesh(core_axis_name="c", subcore_axis_name="s",
                               num_cores=NC)
f32 = jnp.float32
i32 = jnp.int32


def _unpack_lo(w):
    return plsc.bitcast(jnp.left_shift(w, 16), f32)


def _unpack_hi(w):
    return plsc.bitcast(jnp.bitwise_and(w, MSKHI), f32)


def _scatter_prologue(cid, sid, acat, acat_v, zrows, acc):
    pltpu.sync_copy(acat, acat_v)
    aoff = jnp.where(cid == 0, 0, 16).astype(i32)
    pltpu.sync_copy(zrows, acc.at[pl.ds(sid * ROWS_PT, ROWS_PT)])
    plsc.subcore_barrier()
    return aoff


def _make_meta_helpers(ebase, cbase, src2d, dst2d, nrmall, typall,
                       src2_v, dst2_v, nrm2_v, typ2_v, sems_m):
    def issue_meta(sn, st):
        mb = ebase + sn * SCM
        cb = cbase + sn * NSUB
        pltpu.async_copy(src2d.at[pl.ds(cb, NSUB)], src2_v[st], sems_m[st])
        pltpu.async_copy(dst2d.at[pl.ds(cb, NSUB)], dst2_v[st], sems_m[st])
        pltpu.async_copy(nrmall.at[pl.ds(mb, SCM)], nrm2_v[st], sems_m[st])
        pltpu.async_copy(typall.at[pl.ds(mb, SCM)], typ2_v[st], sems_m[st])

    def wait_meta(st):
        for _ in range(4):
            pltpu.make_async_copy(nrmall.at[pl.ds(0, SCM)], nrm2_v[st],
                                  sems_m[st]).wait()

    return issue_meta, wait_meta


def _coeff_super(s2, typ2_v, nrm2_v, cof2_v, acat_v, aoff):
    def kbody(j, _):
        vt = typ2_v[s2][pl.ds(j * 16, 16)]
        vn = nrm2_v[s2][pl.ds(j * 16, 16)]
        va = plsc.load_gather(acat_v, [vt + aoff])
        cof2_v[s2][pl.ds(j * 16, 16)] = vn * va
        return 0

    lax.fori_loop(0, SCM // 16, kbody, 0)


def _scatter_body(esrc, src2d, dst2d, typall, nrmall, acat, zrows, p_out,
                  acc, src2_v, dst2_v, nrm2_v, typ2_v, cof2_v, rows_v,
                  acat_v, sems_m, sems_g, sems_s):
    cid = lax.axis_index("c")
    sid = lax.axis_index("s")
    ebase = cid * NE + sid * MPT
    cbase = cid * (NE // CK) + sid * (MPT // CK)
    aoff = _scatter_prologue(cid, sid, acat, acat_v, zrows, acc)
    issue_meta, wait_meta = _make_meta_helpers(
        ebase, cbase, src2d, dst2d, nrmall, typall,
        src2_v, dst2_v, nrm2_v, typ2_v, sems_m)

    issue_meta(0, 0)

    def bbody(bb, _):
        descs_g = [None] * NB
        descs_s = [None] * NB
        for s2 in range(SPB):
            s = bb * SPB + s2
            wait_meta(s2)
            _coeff_super(s2, typ2_v, nrm2_v, cof2_v, acat_v, aoff)

            def scale(j, b, s2=s2):
                def gbody(g, _):
                    cvec = cof2_v[s2][pl.ds(j * CK + g * 16, 16)]

                    def hbody(hh, _):
                        for k in range(RU):
                            lane = hh * RU + k
                            r = g * 16 + lane
                            cs = jnp.take_along_axis(
                                cvec, jnp.full((16,), lane, dtype=i32),
                                axis=0, mode="promise_in_bounds")
                            for u in range(8):
                                rows_v[b][r, pl.ds(u * 16, 16)] = (
                                    rows_v[b][r, pl.ds(u * 16, 16)] * cs)
                        return 0

                    lax.fori_loop(0, 16 // RU, hbody, 0)
                    return 0

                lax.fori_loop(0, CK // 16, gbody, 0)

            for j in range(NSUB):
                jj = s2 * NSUB + j
                b = jj % NB
                if descs_s[b] is not None:
                    descs_s[b].wait()
                if j == 3:
                    issue_meta(lax.rem(s + 1, NSUP), (s2 + 1) % SPB)
                descs_g[b] = (
                    pltpu.async_copy(
                        esrc.at[src2_v[s2].at[j].at[pl.ds(0, CKH)]],
                        rows_v[b].at[pl.ds(0, CKH)], sems_g[b]),
                    pltpu.async_copy(
                        esrc.at[src2_v[s2].at[j].at[pl.ds(CKH, CKH)]],
                        rows_v[b].at[pl.ds(CKH, CKH)], sems_g[b]),
                )
                if jj >= 1:
                    pj = jj - 1
                    pb = pj % NB
                    ps = pj // NSUB
                    descs_g[pb][0].wait()
                    descs_g[pb][1].wait()
                    scale(pj - ps * NSUB, pb, s2=ps)
                    descs_s[pb] = pltpu.async_copy(
                        rows_v[pb],
                        acc.at[dst2_v[ps].at[pj - ps * NSUB]],
                        sems_s[pb], add=True)
        lj = SPB * NSUB - 1
        lb = lj % NB
        descs_g[lb][0].wait()
        descs_g[lb][1].wait()
        scale(lj - (SPB - 1) * NSUB, lb, s2=SPB - 1)
        descs_s[lb] = pltpu.async_copy(
            rows_v[lb], acc.at[dst2_v[SPB - 1].at[NSUB - 1]], sems_s[lb],
            add=True)
        for b in range(NB):
            if descs_s[b] is not None:
                descs_s[b].wait()
        return 0

    lax.fori_loop(0, NBLK, bbody, 0)
    wait_meta(0)
    plsc.subcore_barrier()
    pltpu.sync_copy(acc.at[pl.ds(sid * ROWS_PT, ROWS_PT)],
                    p_out.at[cid].at[pl.ds(sid * ROWS_PT, ROWS_PT)])


def _scatter_body_packed(esrcp, src2d, dst2d, typall, nrmall, acat, zrows,
                         p_out, acc, src2_v, dst2_v, nrm2_v, typ2_v, cof2_v,
                         rowsb_v, rowsf_v, acat_v, sems_m, sems_g, sems_s):
    cid = lax.axis_index("c")
    sid = lax.axis_index("s")
    ebase = cid * NE + sid * MPT
    cbase = cid * (NE // CK) + sid * (MPT // CK)
    aoff = _scatter_prologue(cid, sid, acat, acat_v, zrows, acc)
    issue_meta, wait_meta = _make_meta_helpers(
        ebase, cbase, src2d, dst2d, nrmall, typall,
        src2_v, dst2_v, nrm2_v, typ2_v, sems_m)

    issue_meta(0, 0)

    def bbody(bb, _):
        descs_g = [None, None]
        descs_s = [None, None]
        for s2 in range(SPB):
            s = bb * SPB + s2
            wait_meta(s2)
            _coeff_super(s2, typ2_v, nrm2_v, cof2_v, acat_v, aoff)

            def scale_p(j, b, ps):
                def gbody(g, _):
                    cvec = cof2_v[ps][pl.ds(j * CK + g * 16, 16)]

                    def hbody(hh, _):
                        for k in range(RUP):
                            lane = hh * RUP + k
                            r = g * 16 + lane
                            cs = jnp.take_along_axis(
                                cvec, jnp.full((16,), lane, dtype=i32),
                                axis=0, mode="promise_in_bounds")
                            for u in range(4):
                                w = rowsb_v[b][r, pl.ds(u * 16, 16)]
                                rowsf_v[b][r, pl.ds(u * 32, 16)] = (
                                    _unpack_lo(w) * cs)
                                rowsf_v[b][r, pl.ds(u * 32 + 16, 16)] = (
                                    _unpack_hi(w) * cs)
                        return 0

                    lax.fori_loop(0, 16 // RUP, hbody, 0)
                    return 0

                lax.fori_loop(0, CK // 16, gbody, 0)

            for j in range(NSUB):
                jj = s2 * NSUB + j
                b = jj % 2
                if descs_s[b] is not None:
                    descs_s[b].wait()
                if j == 3:
                    issue_meta(lax.rem(s + 1, NSUP), (s2 + 1) % SPB)
                descs_g[b] = pltpu.async_copy(
                    esrcp.at[src2_v[s2].at[j]], rowsb_v[b], sems_g[b])
                if jj >= 1:
                    pj = jj - 1
                    pb = pj % 2
                    ps = pj // NSUB
                    descs_g[pb].wait()
                    scale_p(pj - ps * NSUB, pb, ps)
                    descs_s[pb] = pltpu.async_copy(
                        rowsf_v[pb],
                        acc.at[dst2_v[ps].at[pj - ps * NSUB]],
                        sems_s[pb], add=True)
        lj = SPB * NSUB - 1
        lb = lj % 2
        descs_g[lb].wait()
        scale_p(NSUB - 1, lb, SPB - 1)
        descs_s[lb] = pltpu.async_copy(
            rowsf_v[lb], acc.at[dst2_v[SPB - 1].at[NSUB - 1]], sems_s[lb],
            add=True)
        for b in range(2):
            if descs_s[b] is not None:
                descs_s[b].wait()
        return 0

    lax.fori_loop(0, NBLK, bbody, 0)
    wait_meta(0)
    plsc.subcore_barrier()
    pltpu.sync_copy(acc.at[pl.ds(sid * ROWS_PT, ROWS_PT)],
                    p_out.at[cid].at[pl.ds(sid * ROWS_PT, ROWS_PT)])


def _combine_pack_body(p, e_out, bufa, bufb, wbuf, sem):
    cid = lax.axis_index("c")
    sid = lax.axis_index("s")
    wid = sid * NC + cid

    def do(off, n):
        da = pltpu.async_copy(p.at[0].at[pl.ds(off, n)],
                              bufa.at[pl.ds(0, n)], sem)
        db = pltpu.async_copy(p.at[1].at[pl.ds(off, n)],
                              bufb.at[pl.ds(0, n)], sem)
        da.wait(); db.wait()

        def abody(r, _):
            for u in range(4):
                va = (bufa[r, pl.ds(u * 32, 16)]
                      + bufb[r, pl.ds(u * 32, 16)])
                vb = (bufa[r, pl.ds(u * 32 + 16, 16)]
                      + bufb[r, pl.ds(u * 32 + 16, 16)])
                ua = jnp.bitwise_and(
                    jnp.right_shift(plsc.bitcast(va, i32) + 32768, 16),
                    MSK16)
                ub = jnp.bitwise_and(plsc.bitcast(vb, i32) + 32768, MSKHI)
                wbuf[r, pl.ds(u * 16, 16)] = jnp.bitwise_or(ua, ub)
            return 0

        lax.fori_loop(0, n, abody, 0)
        pltpu.sync_copy(wbuf.at[pl.ds(0, n)], e_out.at[pl.ds(off, n)])

    pl.when(wid < 16)(lambda: do(wid * CW_HI, CW_HI))
    pl.when(wid >= 16)(
        lambda: do(16 * CW_HI + (wid - 16) * CW_LO, CW_LO))


def _output_body(oidx_all, emb, e1p, e2p, p3, out_all,
                 oidx_v, orow_v, orowb_v, oacc_v, sem):
    cid = lax.axis_index("c")
    sid = lax.axis_index("s")
    wid = sid * NC + cid

    def obody(q, _):
        ob = wid * OPT + q * OCK
        pltpu.sync_copy(oidx_all.at[pl.ds(ob, OCK)], oidx_v)
        pltpu.async_copy(emb.at[oidx_v], oacc_v, sem).wait()
        for tbl in (e1p, e2p):
            pltpu.async_copy(tbl.at[oidx_v], orowb_v, sem).wait()

            def pbody(j, _):
                for u in range(4):
                    w = orowb_v[j, pl.ds(u * 16, 16)]
                    oacc_v[j, pl.ds(u * 32, 16)] = (
                        oacc_v[j, pl.ds(u * 32, 16)] + _unpack_lo(w))
                    oacc_v[j, pl.ds(u * 32 + 16, 16)] = (
                        oacc_v[j, pl.ds(u * 32 + 16, 16)] + _unpack_hi(w))
                return 0

            lax.fori_loop(0, OCK, pbody, 0)
        for tbl in (p3.at[0], p3.at[1]):
            pltpu.async_copy(tbl.at[oidx_v], orow_v, sem).wait()

            def abody(j, _):
                for u in range(8):
                    oacc_v[j, pl.ds(u * 16, 16)] = (
                        oacc_v[j, pl.ds(u * 16, 16)]
                        + orow_v[j, pl.ds(u * 16, 16)])
                return 0

            lax.fori_loop(0, OCK, abody, 0)

        def sbody(j, _):
            for u in range(8):
                oacc_v[j, pl.ds(u * 16, 16)] = (
                    oacc_v[j, pl.ds(u * 16, 16)] * 0.25)
            return 0

        lax.fori_loop(0, OCK, sbody, 0)
        pltpu.sync_copy(oacc_v, out_all.at[pl.ds(ob, OCK)])
        return 0

    lax.fori_loop(0, OPT // OCK, obody, 0)


_meta_scratch = (
    tuple(pltpu.VMEM((NSUB, CK), i32) for _ in range(SPB)),
    tuple(pltpu.VMEM((NSUB, CK), i32) for _ in range(SPB)),
    tuple(pltpu.VMEM((SCM,), f32) for _ in range(SPB)),
    tuple(pltpu.VMEM((SCM,), i32) for _ in range(SPB)),
    tuple(pltpu.VMEM((SCM,), f32) for _ in range(SPB)),
)

_scatter_call = pl.kernel(
    _scatter_body,
    out_type=jax.ShapeDtypeStruct((NC, N_NODE, D), f32),
    mesh=_mesh,
    scratch_types=(
        pltpu.VMEM_SHARED((N_NODE, D), f32),
        *_meta_scratch,
        tuple(pltpu.VMEM((CK, D), f32) for _ in range(NB)),
        pltpu.VMEM((32,), f32),
        tuple(pltpu.SemaphoreType.DMA for _ in range(SPB)),
        tuple(pltpu.SemaphoreType.DMA for _ in range(NB)),
        tuple(pltpu.SemaphoreType.DMA for _ in range(NB)),
    ),
    compiler_params=_params,
)

_scatter_call_packed = pl.kernel(
    _scatter_body_packed,
    out_type=jax.ShapeDtypeStruct((NC, N_NODE, D), f32),
    mesh=_mesh,
    scratch_types=(
        pltpu.VMEM_SHARED((N_NODE, D), f32),
        *_meta_scratch,
        tuple(pltpu.VMEM((CK, DP), i32) for _ in range(2)),
        tuple(pltpu.VMEM((CK, D), f32) for _ in range(2)),
        pltpu.VMEM((32,), f32),
        tuple(pltpu.SemaphoreType.DMA for _ in range(SPB)),
        tuple(pltpu.SemaphoreType.DMA for _ in range(2)),
        tuple(pltpu.SemaphoreType.DMA for _ in range(2)),
    ),
    compiler_params=_params,
)

_combine_pack_call = pl.kernel(
    _combine_pack_body,
    out_type=jax.ShapeDtypeStruct((N_NODE, DP), i32),
    mesh=_mesh,
    scratch_types=(
        pltpu.VMEM((CW_HI, D), f32),
        pltpu.VMEM((CW_HI, D), f32),
        pltpu.VMEM((CW_HI, DP), i32),
        pltpu.SemaphoreType.DMA,
    ),
    compiler_params=_params,
)

_output_call = pl.kernel(
    _output_body,
    out_type=jax.ShapeDtypeStruct((NOUT, D), f32),
    mesh=_mesh,
    scratch_types=(
        pltpu.VMEM((OCK,), i32),
        pltpu.VMEM((OCK, D), f32),
        pltpu.VMEM((OCK, DP), i32),
        pltpu.VMEM((OCK, D), f32),
        pltpu.SemaphoreType.DMA,
    ),
    compiler_params=_params,
)


@jax.jit
def kernel(user_idx, item_idx, embedding, alpha_head2tail, alpha_tail2head,
           edge_index, edge_type, edge_norm):
    h_arr = edge_index[0]
    t_arr = edge_index[1]
    src2d = jnp.concatenate([t_arr, h_arr]).reshape(NM // CK, CK)
    dst2d = jnp.concatenate([h_arr, t_arr]).reshape(NM // CK, CK)
    typall = jnp.concatenate([edge_type, edge_type])
    nrmall = jnp.concatenate([edge_norm, edge_norm])
    acat = jnp.concatenate([alpha_tail2head, alpha_head2tail])
    oidx_all = jnp.concatenate([user_idx, item_idx + N_USER])
    zrows = jnp.zeros((ROWS_PT, D), f32)

    p1 = _scatter_call(embedding, src2d, dst2d, typall, nrmall, acat, zrows)
    e1p = _combine_pack_call(p1)
    p2 = _scatter_call_packed(e1p, src2d, dst2d, typall, nrmall, acat, zrows)
    e2p = _combine_pack_call(p2)
    p3 = _scatter_call_packed(e2p, src2d, dst2d, typall, nrmall, acat, zrows)
    out_all = _output_call(oidx_all, embedding, e1p, e2p, p3)
    return out_all[:4096], out_all[4096:]


# speedup vs baseline: 1.1239x; 1.1239x over previous
import jax
import jax.numpy as jnp
from jax import lax
from jax.experimental import pallas as pl
from jax.experimental.pallas import tpu as pltpu
from jax.experimental.pallas import tpu_sc as plsc

N_NODE = 10000
N_USER = 6000
D = 128
DP = D // 2
NE = 320000
NM = 2 * NE
NTILE = 16
NC = 2
MPT = NE // NTILE
CK = 80
NSUB = 25
SCM = CK * NSUB
NSUP = MPT // SCM
SPB = 2
NBLK = NSUP // SPB
NB = 3
RU = 4
RUP = 2
CKH = CK // 2
ROWS_PT = N_NODE // NTILE
NOUT = 8192
OPT = NOUT // (NC * NTILE)
OCK = 32
CW_HI = 313
CW_LO = 312
MSK16 = 0xFFFF
MSKHI = -65536

_params = pltpu.CompilerParams(use_tc_tiling_on_sc=False,
                               needs_layout_passes=False)
_mesh = plsc.VectorSubcoreMesh(core_axis_name="c", subcore_axis_name="s",
                               num_cores=NC)
f32 = jnp.float32
i32 = jnp.int32


def _unpack_lo(w):
    return plsc.bitcast(jnp.left_shift(w, 16), f32)


def _unpack_hi(w):
    return plsc.bitcast(jnp.bitwise_and(w, MSKHI), f32)


def _scatter_prologue(cid, sid, acat, acat_v, zrows, acc):
    pltpu.sync_copy(acat, acat_v)
    aoff = jnp.where(cid == 0, 0, 16).astype(i32)
    pltpu.sync_copy(zrows, acc.at[pl.ds(sid * ROWS_PT, ROWS_PT)])
    plsc.subcore_barrier()
    return aoff


def _make_meta_helpers(ebase, cbase, src2d, dst2d, nrmall, typall,
                       src2_v, dst2_v, nrm2_v, typ2_v, sems_m):
    def issue_meta(sn, st):
        mb = ebase + sn * SCM
        cb = cbase + sn * NSUB
        pltpu.async_copy(src2d.at[pl.ds(cb, NSUB)], src2_v[st], sems_m[st])
        pltpu.async_copy(dst2d.at[pl.ds(cb, NSUB)], dst2_v[st], sems_m[st])
        pltpu.async_copy(nrmall.at[pl.ds(mb, SCM)], nrm2_v[st], sems_m[st])
        pltpu.async_copy(typall.at[pl.ds(mb, SCM)], typ2_v[st], sems_m[st])

    def wait_meta(st):
        for _ in range(4):
            pltpu.make_async_copy(nrmall.at[pl.ds(0, SCM)], nrm2_v[st],
                                  sems_m[st]).wait()

    return issue_meta, wait_meta


def _coeff_super(s2, typ2_v, nrm2_v, cof2_v, acat_v, aoff):
    def kbody(j, _):
        vt = typ2_v[s2][pl.ds(j * 16, 16)]
        vn = nrm2_v[s2][pl.ds(j * 16, 16)]
        va = plsc.load_gather(acat_v, [vt + aoff])
        cof2_v[s2][pl.ds(j * 16, 16)] = vn * va
        return 0

    lax.fori_loop(0, SCM // 16, kbody, 0)


def _scatter_body(esrc, src2d, dst2d, typall, nrmall, acat, zrows, p_out,
                  acc, src2_v, dst2_v, nrm2_v, typ2_v, cof2_v, rows_v,
                  acat_v, sems_m, sems_g, sems_s):
    cid = lax.axis_index("c")
    sid = lax.axis_index("s")
    ebase = cid * NE + sid * MPT
    cbase = cid * (NE // CK) + sid * (MPT // CK)
    aoff = _scatter_prologue(cid, sid, acat, acat_v, zrows, acc)
    issue_meta, wait_meta = _make_meta_helpers(
        ebase, cbase, src2d, dst2d, nrmall, typall,
        src2_v, dst2_v, nrm2_v, typ2_v, sems_m)

    issue_meta(0, 0)

    def bbody(bb, _):
        descs_g = [None] * NB
        descs_s = [None] * NB
        for s2 in range(SPB):
            s = bb * SPB + s2
            wait_meta(s2)
            _coeff_super(s2, typ2_v, nrm2_v, cof2_v, acat_v, aoff)

            def scale(j, b, s2=s2):
                def gbody(g, _):
                    cvec = cof2_v[s2][pl.ds(j * CK + g * 16, 16)]

                    def hbody(hh, _):
                        for k in range(RU):
                            lane = hh * RU + k
                            r = g * 16 + lane
                            cs = jnp.take_along_axis(
                                cvec, jnp.full((16,), lane, dtype=i32),
                                axis=0, mode="promise_in_bounds")
                            for u in range(8):
                                rows_v[b][r, pl.ds(u * 16, 16)] = (
                                    rows_v[b][r, pl.ds(u * 16, 16)] * cs)
                        return 0

                    lax.fori_loop(0, 16 // RU, hbody, 0)
                    return 0

                lax.fori_loop(0, CK // 16, gbody, 0)

            for j in range(NSUB):
                jj = s2 * NSUB + j
                b = jj % NB
                if descs_s[b] is not None:
                    descs_s[b].wait()
                if j == 3:
                    issue_meta(lax.rem(s + 1, NSUP), (s2 + 1) % SPB)
                descs_g[b] = (
                    pltpu.async_copy(
                        esrc.at[src2_v[s2].at[j].at[pl.ds(0, CKH)]],
                        rows_v[b].at[pl.ds(0, CKH)], sems_g[b]),
                    pltpu.async_copy(
                        esrc.at[src2_v[s2].at[j].at[pl.ds(CKH, CKH)]],
                        rows_v[b].at[pl.ds(CKH, CKH)], sems_g[b]),
                )
                if jj >= 1:
                    pj = jj - 1
                    pb = pj % NB
                    ps = pj // NSUB
                    descs_g[pb][0].wait()
                    descs_g[pb][1].wait()
                    scale(pj - ps * NSUB, pb, s2=ps)
                    descs_s[pb] = pltpu.async_copy(
                        rows_v[pb],
                        acc.at[dst2_v[ps].at[pj - ps * NSUB]],
                        sems_s[pb], add=True)
        lj = SPB * NSUB - 1
        lb = lj % NB
        descs_g[lb][0].wait()
        descs_g[lb][1].wait()
        scale(lj - (SPB - 1) * NSUB, lb, s2=SPB - 1)
        descs_s[lb] = pltpu.async_copy(
            rows_v[lb], acc.at[dst2_v[SPB - 1].at[NSUB - 1]], sems_s[lb],
            add=True)
        for b in range(NB):
            if descs_s[b] is not None:
                descs_s[b].wait()
        return 0

    lax.fori_loop(0, NBLK, bbody, 0)
    wait_meta(0)
    plsc.subcore_barrier()
    pltpu.sync_copy(acc.at[pl.ds(sid * ROWS_PT, ROWS_PT)],
                    p_out.at[cid].at[pl.ds(sid * ROWS_PT, ROWS_PT)])


def _scatter_body_packed(esrcp, src2d, dst2d, typall, nrmall, acat, zrows,
                         p_out, acc, src2_v, dst2_v, nrm2_v, typ2_v, cof2_v,
                         rowsb_v, rowsf_v, acat_v, sems_m, sems_g, sems_s):
    cid = lax.axis_index("c")
    sid = lax.axis_index("s")
    ebase = cid * NE + sid * MPT
    cbase = cid * (NE // CK) + sid * (MPT // CK)
    aoff = _scatter_prologue(cid, sid, acat, acat_v, zrows, acc)
    issue_meta, wait_meta = _make_meta_helpers(
        ebase, cbase, src2d, dst2d, nrmall, typall,
        src2_v, dst2_v, nrm2_v, typ2_v, sems_m)

    def sbody(s, _):
        issue_meta(s, 0)
        wait_meta(0)
        _coeff_super(0, typ2_v, nrm2_v, cof2_v, acat_v, aoff)

        def scale_p(j, bg, bf):
            def gbody(g, _):
                cvec = cof2_v[0][pl.ds(j * CK + g * 16, 16)]

                def hbody(hh, _):
                    for k in range(RUP):
                        lane = hh * RUP + k
                        r = g * 16 + lane
                        cs = jnp.take_along_axis(
                            cvec, jnp.full((16,), lane, dtype=i32),
                            axis=0, mode="promise_in_bounds")
                        for u in range(4):
                            w = rowsb_v[bg][r, pl.ds(u * 16, 16)]
                            rowsf_v[bf][r, pl.ds(u * 32, 16)] = (
                                _unpack_lo(w) * cs)
                            rowsf_v[bf][r, pl.ds(u * 32 + 16, 16)] = (
                                _unpack_hi(w) * cs)
                    return 0

                lax.fori_loop(0, 16 // RUP, hbody, 0)
                return 0

            lax.fori_loop(0, CK // 16, gbody, 0)

        descs_g = [None, None]
        descs_s = [None] * NB
        for j in range(NSUB):
            bg = j % 2
            bf = j % NB
            if descs_s[bf] is not None:
                descs_s[bf].wait()
            descs_g[bg] = pltpu.async_copy(
                esrcp.at[src2_v[0].at[j]], rowsb_v[bg], sems_g[bg])
            if j >= 1:
                pg = (j - 1) % 2
                pf = (j - 1) % NB
                descs_g[pg].wait()
                scale_p(j - 1, pg, pf)
                descs_s[pf] = pltpu.async_copy(
                    rowsf_v[pf],
                    acc.at[dst2_v[0].at[j - 1]],
                    sems_s[pf], add=True)
        lg = (NSUB - 1) % 2
        lf = (NSUB - 1) % NB
        descs_g[lg].wait()
        scale_p(NSUB - 1, lg, lf)
        descs_s[lf] = pltpu.async_copy(
            rowsf_v[lf], acc.at[dst2_v[0].at[NSUB - 1]], sems_s[lf],
            add=True)
        for b in range(NB):
            if descs_s[b] is not None:
                descs_s[b].wait()
        return 0

    lax.fori_loop(0, NSUP, sbody, 0)
    plsc.subcore_barrier()
    pltpu.sync_copy(acc.at[pl.ds(sid * ROWS_PT, ROWS_PT)],
                    p_out.at[cid].at[pl.ds(sid * ROWS_PT, ROWS_PT)])


def _combine_pack_body(p, e_out, bufa, bufb, wbuf, sem):
    cid = lax.axis_index("c")
    sid = lax.axis_index("s")
    wid = sid * NC + cid

    def do(off, n):
        da = pltpu.async_copy(p.at[0].at[pl.ds(off, n)],
                              bufa.at[pl.ds(0, n)], sem)
        db = pltpu.async_copy(p.at[1].at[pl.ds(off, n)],
                              bufb.at[pl.ds(0, n)], sem)
        da.wait(); db.wait()

        def abody(r, _):
            for u in range(4):
                va = (bufa[r, pl.ds(u * 32, 16)]
                      + bufb[r, pl.ds(u * 32, 16)])
                vb = (bufa[r, pl.ds(u * 32 + 16, 16)]
                      + bufb[r, pl.ds(u * 32 + 16, 16)])
                ua = jnp.bitwise_and(
                    jnp.right_shift(plsc.bitcast(va, i32) + 32768, 16),
                    MSK16)
                ub = jnp.bitwise_and(plsc.bitcast(vb, i32) + 32768, MSKHI)
                wbuf[r, pl.ds(u * 16, 16)] = jnp.bitwise_or(ua, ub)
            return 0

        lax.fori_loop(0, n, abody, 0)
        pltpu.sync_copy(wbuf.at[pl.ds(0, n)], e_out.at[pl.ds(off, n)])

    pl.when(wid < 16)(lambda: do(wid * CW_HI, CW_HI))
    pl.when(wid >= 16)(
        lambda: do(16 * CW_HI + (wid - 16) * CW_LO, CW_LO))


def _output_body(oidx_all, emb, e1p, e2p, p3, out_all,
                 oidx_v, orow_v, orowb_v, oacc_v, sem):
    cid = lax.axis_index("c")
    sid = lax.axis_index("s")
    wid = sid * NC + cid

    def obody(q, _):
        ob = wid * OPT + q * OCK
        pltpu.sync_copy(oidx_all.at[pl.ds(ob, OCK)], oidx_v)
        pltpu.async_copy(emb.at[oidx_v], oacc_v, sem).wait()
        for tbl in (e1p, e2p):
            pltpu.async_copy(tbl.at[oidx_v], orowb_v, sem).wait()

            def pbody(j, _):
                for u in range(4):
                    w = orowb_v[j, pl.ds(u * 16, 16)]
                    oacc_v[j, pl.ds(u * 32, 16)] = (
                        oacc_v[j, pl.ds(u * 32, 16)] + _unpack_lo(w))
                    oacc_v[j, pl.ds(u * 32 + 16, 16)] = (
                        oacc_v[j, pl.ds(u * 32 + 16, 16)] + _unpack_hi(w))
                return 0

            lax.fori_loop(0, OCK, pbody, 0)
        for tbl in (p3.at[0], p3.at[1]):
            pltpu.async_copy(tbl.at[oidx_v], orow_v, sem).wait()

            def abody(j, _):
                for u in range(8):
                    oacc_v[j, pl.ds(u * 16, 16)] = (
                        oacc_v[j, pl.ds(u * 16, 16)]
                        + orow_v[j, pl.ds(u * 16, 16)])
                return 0

            lax.fori_loop(0, OCK, abody, 0)

        def sbody(j, _):
            for u in range(8):
                oacc_v[j, pl.ds(u * 16, 16)] = (
                    oacc_v[j, pl.ds(u * 16, 16)] * 0.25)
            return 0

        lax.fori_loop(0, OCK, sbody, 0)
        pltpu.sync_copy(oacc_v, out_all.at[pl.ds(ob, OCK)])
        return 0

    lax.fori_loop(0, OPT // OCK, obody, 0)


def _meta_scratch(nsets):
    return (
        tuple(pltpu.VMEM((NSUB, CK), i32) for _ in range(nsets)),
        tuple(pltpu.VMEM((NSUB, CK), i32) for _ in range(nsets)),
        tuple(pltpu.VMEM((SCM,), f32) for _ in range(nsets)),
        tuple(pltpu.VMEM((SCM,), i32) for _ in range(nsets)),
        tuple(pltpu.VMEM((SCM,), f32) for _ in range(nsets)),
    )

_scatter_call = pl.kernel(
    _scatter_body,
    out_type=jax.ShapeDtypeStruct((NC, N_NODE, D), f32),
    mesh=_mesh,
    scratch_types=(
        pltpu.VMEM_SHARED((N_NODE, D), f32),
        *_meta_scratch(SPB),
        tuple(pltpu.VMEM((CK, D), f32) for _ in range(NB)),
        pltpu.VMEM((32,), f32),
        tuple(pltpu.SemaphoreType.DMA for _ in range(SPB)),
        tuple(pltpu.SemaphoreType.DMA for _ in range(NB)),
        tuple(pltpu.SemaphoreType.DMA for _ in range(NB)),
    ),
    compiler_params=_params,
)

_scatter_call_packed = pl.kernel(
    _scatter_body_packed,
    out_type=jax.ShapeDtypeStruct((NC, N_NODE, D), f32),
    mesh=_mesh,
    scratch_types=(
        pltpu.VMEM_SHARED((N_NODE, D), f32),
        *_meta_scratch(1),
        tuple(pltpu.VMEM((CK, DP), i32) for _ in range(2)),
        tuple(pltpu.VMEM((CK, D), f32) for _ in range(NB)),
        pltpu.VMEM((32,), f32),
        tuple(pltpu.SemaphoreType.DMA for _ in range(1)),
        tuple(pltpu.SemaphoreType.DMA for _ in range(2)),
        tuple(pltpu.SemaphoreType.DMA for _ in range(NB)),
    ),
    compiler_params=_params,
)

_combine_pack_call = pl.kernel(
    _combine_pack_body,
    out_type=jax.ShapeDtypeStruct((N_NODE, DP), i32),
    mesh=_mesh,
    scratch_types=(
        pltpu.VMEM((CW_HI, D), f32),
        pltpu.VMEM((CW_HI, D), f32),
        pltpu.VMEM((CW_HI, DP), i32),
        pltpu.SemaphoreType.DMA,
    ),
    compiler_params=_params,
)

_output_call = pl.kernel(
    _output_body,
    out_type=jax.ShapeDtypeStruct((NOUT, D), f32),
    mesh=_mesh,
    scratch_types=(
        pltpu.VMEM((OCK,), i32),
        pltpu.VMEM((OCK, D), f32),
        pltpu.VMEM((OCK, DP), i32),
        pltpu.VMEM((OCK, D), f32),
        pltpu.SemaphoreType.DMA,
    ),
    compiler_params=_params,
)


@jax.jit
def kernel(user_idx, item_idx, embedding, alpha_head2tail, alpha_tail2head,
           edge_index, edge_type, edge_norm):
    h_arr = edge_index[0]
    t_arr = edge_index[1]
    src2d = jnp.concatenate([t_arr, h_arr]).reshape(NM // CK, CK)
    dst2d = jnp.concatenate([h_arr, t_arr]).reshape(NM // CK, CK)
    typall = jnp.concatenate([edge_type, edge_type])
    nrmall = jnp.concatenate([edge_norm, edge_norm])
    acat = jnp.concatenate([alpha_tail2head, alpha_head2tail])
    oidx_all = jnp.concatenate([user_idx, item_idx + N_USER])
    zrows = jnp.zeros((ROWS_PT, D), f32)

    p1 = _scatter_call(embedding, src2d, dst2d, typall, nrmall, acat, zrows)
    e1p = _combine_pack_call(p1)
    p2 = _scatter_call_packed(e1p, src2d, dst2d, typall, nrmall, acat, zrows)
    e2p = _combine_pack_call(p2)
    p3 = _scatter_call_packed(e2p, src2d, dst2d, typall, nrmall, acat, zrows)
    out_all = _output_call(oidx_all, embedding, e1p, e2p, p3)
    return out_all[:4096], out_all[4096:]


# speedup vs baseline: 1.8096x; 1.6100x over previous
import jax
import jax.numpy as jnp
from jax import lax
from jax.experimental import pallas as pl
from jax.experimental.pallas import tpu as pltpu
from jax.experimental.pallas import tpu_sc as plsc

N_NODE = 10000
N_USER = 6000
D = 128
NE = 320000
NM = 2 * NE
NTILE = 16
NC = 2
MPT = NE // NTILE
CK = 80
NSUB = 25
SCM = CK * NSUB
NSUP = MPT // SCM
SPB = 2
NBLK = NSUP // SPB
NB = 3
CKH = CK // 2
RU = 4
ROWS_PT = N_NODE // NTILE
NOUT = 8192
OPT = NOUT // (NC * NTILE)
OCK = 32
CW_HI = 313
CW_LO = 312

_params = pltpu.CompilerParams(use_tc_tiling_on_sc=False,
                               needs_layout_passes=False)
_mesh = plsc.VectorSubcoreMesh(core_axis_name="c", subcore_axis_name="s",
                               num_cores=NC)
f32 = jnp.float32
i32 = jnp.int32


def _scatter_body(esrc, src2d, dst2d, typall, nrmall, acat, zrows, p_out,
                  acc, src2_v, dst2_v, nrm2_v, typ2_v, cof2_v, rows_v,
                  acat_v, sems_m, sems_g, sems_s):
    cid = lax.axis_index("c")
    sid = lax.axis_index("s")
    ebase = cid * NE + sid * MPT
    cbase = cid * (NE // CK) + sid * (MPT // CK)

    pltpu.sync_copy(acat, acat_v)
    aoff = jnp.where(cid == 0, 0, 16).astype(i32)

    pltpu.sync_copy(zrows, acc.at[pl.ds(sid * ROWS_PT, ROWS_PT)])
    plsc.subcore_barrier()

    def issue_meta(sn, st):
        mb = ebase + sn * SCM
        cb = cbase + sn * NSUB
        pltpu.async_copy(src2d.at[pl.ds(cb, NSUB)], src2_v[st], sems_m[st])
        pltpu.async_copy(dst2d.at[pl.ds(cb, NSUB)], dst2_v[st], sems_m[st])
        pltpu.async_copy(nrmall.at[pl.ds(mb, SCM)], nrm2_v[st], sems_m[st])
        pltpu.async_copy(typall.at[pl.ds(mb, SCM)], typ2_v[st], sems_m[st])

    def wait_meta(st):
        for _ in range(4):
            pltpu.make_async_copy(nrmall.at[pl.ds(0, SCM)], nrm2_v[st],
                                  sems_m[st]).wait()

    issue_meta(0, 0)

    def bbody(bb, _):
        descs_g = [None] * NB
        descs_s = [None] * NB
        for s2 in range(SPB):
            s = bb * SPB + s2
            wait_meta(s2)

            def kbody(j, _, s2=s2):
                vt = typ2_v[s2][pl.ds(j * 16, 16)]
                vn = nrm2_v[s2][pl.ds(j * 16, 16)]
                va = plsc.load_gather(acat_v, [vt + aoff])
                cof2_v[s2][pl.ds(j * 16, 16)] = vn * va
                return 0

            lax.fori_loop(0, SCM // 16, kbody, 0)

            def scale(j, b, s2=s2):
                def gbody(g, _):
                    cvec = cof2_v[s2][pl.ds(j * CK + g * 16, 16)]

                    def hbody(hh, _):
                        for k in range(RU):
                            lane = hh * RU + k
                            r = g * 16 + lane
                            cs = jnp.take_along_axis(
                                cvec, jnp.full((16,), lane, dtype=i32),
                                axis=0, mode="promise_in_bounds")
                            for u in range(8):
                                rows_v[b][r, pl.ds(u * 16, 16)] = (
                                    rows_v[b][r, pl.ds(u * 16, 16)] * cs)
                        return 0

                    lax.fori_loop(0, 16 // RU, hbody, 0)
                    return 0

                lax.fori_loop(0, CK // 16, gbody, 0)

            for j in range(NSUB):
                jj = s2 * NSUB + j
                b = jj % NB
                if descs_s[b] is not None:
                    descs_s[b].wait()
                if j == 3:
                    issue_meta(lax.rem(s + 1, NSUP), (s2 + 1) % SPB)
                descs_g[b] = (
                    pltpu.async_copy(
                        esrc.at[src2_v[s2].at[j].at[pl.ds(0, CKH)]],
                        rows_v[b].at[pl.ds(0, CKH)], sems_g[b]),
                    pltpu.async_copy(
                        esrc.at[src2_v[s2].at[j].at[pl.ds(CKH, CKH)]],
                        rows_v[b].at[pl.ds(CKH, CKH)], sems_g[b]),
                )
                if jj >= 1:
                    pj = jj - 1
                    pb = pj % NB
                    ps = pj // NSUB
                    descs_g[pb][0].wait()
                    descs_g[pb][1].wait()
                    scale(pj - ps * NSUB, pb, s2=ps)
                    descs_s[pb] = pltpu.async_copy(
                        rows_v[pb],
                        acc.at[dst2_v[ps].at[pj - ps * NSUB]],
                        sems_s[pb], add=True)
        lj = SPB * NSUB - 1
        lb = lj % NB
        descs_g[lb][0].wait()
        descs_g[lb][1].wait()
        scale(lj - (SPB - 1) * NSUB, lb, s2=SPB - 1)
        descs_s[lb] = pltpu.async_copy(
            rows_v[lb], acc.at[dst2_v[SPB - 1].at[NSUB - 1]], sems_s[lb],
            add=True)
        for b in range(NB):
            if descs_s[b] is not None:
                descs_s[b].wait()
        return 0

    lax.fori_loop(0, NBLK, bbody, 0)
    wait_meta(0)
    plsc.subcore_barrier()
    pltpu.sync_copy(acc.at[pl.ds(sid * ROWS_PT, ROWS_PT)],
                    p_out.at[cid].at[pl.ds(sid * ROWS_PT, ROWS_PT)])


def _combine_body(p, e_out, bufa, bufb, sem):
    cid = lax.axis_index("c")
    sid = lax.axis_index("s")
    wid = sid * NC + cid

    def do(off, n):
        da = pltpu.async_copy(p.at[0].at[pl.ds(off, n)],
                              bufa.at[pl.ds(0, n)], sem)
        db = pltpu.async_copy(p.at[1].at[pl.ds(off, n)],
                              bufb.at[pl.ds(0, n)], sem)
        da.wait(); db.wait()

        def abody(r, _):
            for u in range(8):
                bufa[r, pl.ds(u * 16, 16)] = (
                    bufa[r, pl.ds(u * 16, 16)] + bufb[r, pl.ds(u * 16, 16)])
            return 0

        lax.fori_loop(0, n, abody, 0)
        pltpu.sync_copy(bufa.at[pl.ds(0, n)], e_out.at[pl.ds(off, n)])

    pl.when(wid < 16)(lambda: do(wid * CW_HI, CW_HI))
    pl.when(wid >= 16)(
        lambda: do(16 * CW_HI + (wid - 16) * CW_LO, CW_LO))


def _output_body(oidx_all, emb, e1, e2, p3, out_all,
                 oidx_v, orow_v, oacc_v, sem):
    cid = lax.axis_index("c")
    sid = lax.axis_index("s")
    wid = sid * NC + cid

    def obody(q, _):
        ob = wid * OPT + q * OCK
        pltpu.sync_copy(oidx_all.at[pl.ds(ob, OCK)], oidx_v)
        pltpu.async_copy(emb.at[oidx_v], oacc_v, sem).wait()
        for tbl in (e1, e2, p3.at[0], p3.at[1]):
            pltpu.async_copy(tbl.at[oidx_v], orow_v, sem).wait()

            def abody(j, _):
                for u in range(8):
                    oacc_v[j, pl.ds(u * 16, 16)] = (
                        oacc_v[j, pl.ds(u * 16, 16)]
                        + orow_v[j, pl.ds(u * 16, 16)])
                return 0

            lax.fori_loop(0, OCK, abody, 0)

        def sbody(j, _):
            for u in range(8):
                oacc_v[j, pl.ds(u * 16, 16)] = (
                    oacc_v[j, pl.ds(u * 16, 16)] * 0.25)
            return 0

        lax.fori_loop(0, OCK, sbody, 0)
        pltpu.sync_copy(oacc_v, out_all.at[pl.ds(ob, OCK)])
        return 0

    lax.fori_loop(0, OPT // OCK, obody, 0)


_scatter_call = pl.kernel(
    _scatter_body,
    out_type=jax.ShapeDtypeStruct((NC, N_NODE, D), f32),
    mesh=_mesh,
    scratch_types=(
        pltpu.VMEM_SHARED((N_NODE, D), f32),
        tuple(pltpu.VMEM((NSUB, CK), i32) for _ in range(SPB)),
        tuple(pltpu.VMEM((NSUB, CK), i32) for _ in range(SPB)),
        tuple(pltpu.VMEM((SCM,), f32) for _ in range(SPB)),
        tuple(pltpu.VMEM((SCM,), i32) for _ in range(SPB)),
        tuple(pltpu.VMEM((SCM,), f32) for _ in range(SPB)),
        tuple(pltpu.VMEM((CK, D), f32) for _ in range(NB)),
        pltpu.VMEM((32,), f32),
        tuple(pltpu.SemaphoreType.DMA for _ in range(SPB)),
        tuple(pltpu.SemaphoreType.DMA for _ in range(NB)),
        tuple(pltpu.SemaphoreType.DMA for _ in range(NB)),
    ),
    compiler_params=_params,
)

_combine_call = pl.kernel(
    _combine_body,
    out_type=jax.ShapeDtypeStruct((N_NODE, D), f32),
    mesh=_mesh,
    scratch_types=(
        pltpu.VMEM((CW_HI, D), f32),
        pltpu.VMEM((CW_HI, D), f32),
        pltpu.SemaphoreType.DMA,
    ),
    compiler_params=_params,
)

_output_call = pl.kernel(
    _output_body,
    out_type=jax.ShapeDtypeStruct((NOUT, D), f32),
    mesh=_mesh,
    scratch_types=(
        pltpu.VMEM((OCK,), i32),
        pltpu.VMEM((OCK, D), f32),
        pltpu.VMEM((OCK, D), f32),
        pltpu.SemaphoreType.DMA,
    ),
    compiler_params=_params,
)


@jax.jit
def kernel(user_idx, item_idx, embedding, alpha_head2tail, alpha_tail2head,
           edge_index, edge_type, edge_norm):
    h_arr = edge_index[0]
    t_arr = edge_index[1]
    src2d = jnp.concatenate([t_arr, h_arr]).reshape(NM // CK, CK)
    dst2d = jnp.concatenate([h_arr, t_arr]).reshape(NM // CK, CK)
    typall = jnp.concatenate([edge_type, edge_type])
    nrmall = jnp.concatenate([edge_norm, edge_norm])
    acat = jnp.concatenate([alpha_tail2head, alpha_head2tail])
    oidx_all = jnp.concatenate([user_idx, item_idx + N_USER])
    zrows = jnp.zeros((ROWS_PT, D), f32)

    p1 = _scatter_call(embedding, src2d, dst2d, typall, nrmall, acat, zrows)
    e1 = _combine_call(p1)
    p2 = _scatter_call(e1, src2d, dst2d, typall, nrmall, acat, zrows)
    e2 = _combine_call(p2)
    p3 = _scatter_call(e2, src2d, dst2d, typall, nrmall, acat, zrows)
    out_all = _output_call(oidx_all, embedding, e1, e2, p3)
    return out_all[:4096], out_all[4096:]
